# baseline (speedup 1.0000x reference)
import functools

import jax
import jax.numpy as jnp
from jax.experimental import pallas as pl
from jax.experimental.pallas import tpu as pltpu

_LN_EPS = 1e-5
_VMEM_LIMIT = 56 * 1024 * 1024
_CENTRE = 4
_HEADS = 8


def _layernorm(x, g, b):
    mu = jnp.mean(x, axis=-1, keepdims=True)
    var = jnp.mean(jnp.square(x - mu), axis=-1, keepdims=True)
    return (x - mu) * jax.lax.rsqrt(var + _LN_EPS) * g + b


def _gelu(x):
    a1, a2, a3, a4, a5 = (0.254829592, -0.284496736, 1.421413741,
                          -1.453152027, 1.061405429)
    pc = 0.3275911
    z = x * 0.7071067811865476
    az = jnp.abs(z)
    t = pl.reciprocal(1.0 + pc * az, approx=True)
    poly = ((((a5 * t + a4) * t + a3) * t + a2) * t + a1) * t
    erf_abs = 1.0 - poly * jnp.exp(-az * az)
    erf = jnp.where(z < 0.0, -erf_abs, erf_abs)
    return 0.5 * x * (1.0 + erf)


def _dot_tb(x, w):
    return jax.lax.dot_general(x, w, (((1,), (1,)), ((), ())),
                               preferred_element_type=jnp.float32)


def _inter_block(px, pos_ref, wpe, wq, wkv, wo,
                 bpe, g_in, b_ln, bq, bkv_b, bo,
                 *, n_nb, centre, heads):
    PN = px.shape[1]
    E = wpe.shape[0]
    P = PN // n_nb
    d = E // heads
    scale = 1.0 / (d ** 0.5)

    lane = jax.lax.broadcasted_iota(jnp.int32, (E, heads), 0)
    head = jax.lax.broadcasted_iota(jnp.int32, (E, heads), 1)
    hsel = (lane // d == head).astype(jnp.float32)

    emb = jax.lax.dot_general(px.astype(jnp.bfloat16), wpe,
                              (((0,), (1,)), ((), ())),
                              preferred_element_type=jnp.float32) + bpe
    x3 = emb.astype(jnp.bfloat16).reshape(n_nb, P, E) \
        + pos_ref[0, :n_nb].astype(jnp.bfloat16)[:, None, :]

    xf = x3.astype(jnp.float32).reshape(PN, E)
    xn = _layernorm(xf, g_in, b_ln)

    kv = _dot_tb(xn.astype(jnp.bfloat16), wkv) + bkv_b
    xc = xn.reshape(n_nb, P, E)[centre]
    q = _dot_tb(xc.astype(jnp.bfloat16), wq) + bq

    kv3 = kv.reshape(n_nb, P, 2 * E)
    k3 = kv3[:, :, :E]
    v3 = kv3[:, :, E:]

    s_all = q[None, :, :] * k3
    s_h = jnp.dot(s_all.reshape(PN, E), hsel,
                  preferred_element_type=jnp.float32) * scale
    s_h = s_h.reshape(n_nb, P, heads)

    m = jnp.max(s_h, axis=0, keepdims=True)
    p = jnp.exp(s_h - m)
    den = jnp.sum(p, axis=0, keepdims=True)
    p = p * pl.reciprocal(den, approx=True)

    p_full = jax.lax.dot_general(p.reshape(PN, heads), hsel,
                                 (((1,), (1,)), ((), ())),
                                 preferred_element_type=jnp.float32)
    ctx = jnp.sum(p_full.reshape(n_nb, P, E) * v3, axis=0)

    out = _dot_tb(ctx.astype(jnp.bfloat16), wo) + bo
    out = out + xf.reshape(n_nb, P, E)[centre]
    return out.astype(jnp.bfloat16)


def _full_kernel(px_ref, pos_ref, wpe_ref, w_in_ref, wo_ref,
                 w_in2_ref, wo2_ref, w1_ref, w2_ref, s_ref, o_ref,
                 *, n_nb, centre, heads, bb):
    bf16 = jnp.bfloat16
    E = wpe_ref.shape[0]
    s = s_ref[...]
    wpe = wpe_ref[...].astype(bf16)
    wq = w_in_ref[:E].astype(bf16)
    wkv = w_in_ref[E:].astype(bf16)
    wo = wo_ref[...].astype(bf16)
    w_in2 = w_in2_ref[...].astype(bf16)
    wo2 = wo2_ref[...].astype(bf16)
    w1 = w1_ref[...].astype(bf16)
    w2 = w2_ref[...].astype(bf16)
    bkv_b = jnp.concatenate([s[4:5], s[5:6]], axis=1)
    b_in2 = jnp.concatenate([s[7:8], s[8:9], s[9:10]], axis=1)
    for i in range(bb):
        centre_rows = _inter_block(
            px_ref[i], pos_ref, wpe, wq, wkv, wo,
            s[0:1], s[1:2], s[2:3], s[3:4], bkv_b, s[6:7],
            n_nb=n_nb, centre=centre, heads=heads)
        _intra_block(centre_rows, w_in2, wo2, w1, w2,
                     b_in2, s[10:11], s[11:12], s[12:13], s[13:14],
                     s[14:15], s[15:16], s[16:17], s[17:20], o_ref, i,
                     heads=heads)


def _intra_block(x_in, w_in2, wo2, w1, w2,
                 b_in, bo, ffg, ffb, b1, b2, fg, fb, we, o_ref, oi, *, heads):
    x = x_in.astype(jnp.float32)
    P, E = x.shape
    d = E // heads
    scale = 1.0 / (d ** 0.5)

    qkv = _dot_tb(x.astype(jnp.bfloat16), w_in2) + b_in

    ctx = []
    for h in range(heads):
        lo = h * d
        q_h = qkv[:, lo:lo + d].astype(jnp.bfloat16)
        k_h = qkv[:, E + lo:E + lo + d].astype(jnp.bfloat16)
        v_h = qkv[:, 2 * E + lo:2 * E + lo + d].astype(jnp.bfloat16)
        s = jax.lax.dot_general(q_h, k_h, (((1,), (1,)), ((), ())),
                                preferred_element_type=jnp.float32) * scale
        m = jnp.max(s, axis=-1, keepdims=True)
        p = jnp.exp(s - m)
        den = jnp.sum(p, axis=-1, keepdims=True)
        attn = p * pl.reciprocal(den, approx=True)
        ctx.append(jnp.dot(attn.astype(jnp.bfloat16), v_h,
                           preferred_element_type=jnp.float32))
    ctx = jnp.concatenate(ctx, axis=-1)

    att = _dot_tb(ctx.astype(jnp.bfloat16), wo2) + bo
    y = att + x

    yn = _layernorm(y, ffg, ffb)
    h1 = _dot_tb(yn.astype(jnp.bfloat16), w1) + b1
    h1 = _gelu(h1)
    h2 = _dot_tb(h1.astype(jnp.bfloat16), w2) + b2
    z = (h2 + y).astype(jnp.bfloat16).astype(jnp.float32)

    zn = _layernorm(z, fg, fb)
    we8 = jnp.concatenate(
        [we.astype(jnp.bfloat16),
         jnp.zeros((8 - we.shape[0], E), jnp.bfloat16)], axis=0)
    out_t = jax.lax.dot_general(we8, zn.astype(jnp.bfloat16),
                                (((1,), (1,)), ((), ())),
                                preferred_element_type=jnp.float32)
    o_ref[oi] = out_t


def kernel(img, pixel_embedding_w, pixel_embedding_b, pos_embedding,
           final_ln_g, final_ln_b, embedding2pixel_w, embedding2pixel_b,
           l0_inter_ln_g, l0_inter_ln_b, l0_inter_att_in_w, l0_inter_att_in_b,
           l0_inter_att_out_w, l0_inter_att_out_b,
           l0_intra_att_in_w, l0_intra_att_in_b, l0_intra_att_out_w,
           l0_intra_att_out_b, l0_ff_ln_g, l0_ff_ln_b, l0_ff_w1, l0_ff_b1,
           l0_ff_w2, l0_ff_b2):
    B, C, N, Himg, Wimg = img.shape
    P = Himg * Wimg
    E = pos_embedding.shape[-1]
    heads = _HEADS
    H = l0_ff_w1.shape[0]

    f32, bf16 = jnp.float32, jnp.bfloat16

    px = img.reshape(B, C, N * P)

    small = jnp.concatenate([
        pixel_embedding_b.reshape(1, E),
        l0_inter_ln_g.reshape(1, E),
        l0_inter_ln_b.reshape(1, E),
        l0_inter_att_in_b.reshape(3, E),
        l0_inter_att_out_b.reshape(1, E),
        l0_intra_att_in_b.reshape(3, E),
        l0_intra_att_out_b.reshape(1, E),
        l0_ff_ln_g.reshape(1, E),
        l0_ff_ln_b.reshape(1, E),
        l0_ff_b1.reshape(1, H),
        l0_ff_b2.reshape(1, E),
        final_ln_g.reshape(1, E),
        final_ln_b.reshape(1, E),
        embedding2pixel_w,
        jnp.zeros((4, E), f32),
    ], axis=0)

    _const = lambda b: (0, 0)
    bb = 4 if B % 4 == 0 else (2 if B % 2 == 0 else 1)
    kern = functools.partial(_full_kernel, n_nb=N, centre=_CENTRE,
                             heads=heads, bb=bb)
    y = pl.pallas_call(
        kern,
        out_shape=jax.ShapeDtypeStruct((B, 8, P), f32),
        grid_spec=pltpu.PrefetchScalarGridSpec(
            num_scalar_prefetch=0,
            grid=(B // bb,),
            in_specs=[
                pl.BlockSpec((bb, C, N * P), lambda b: (b, 0, 0)),
                pl.BlockSpec((1, P, E), lambda b: (0, 0, 0)),
                pl.BlockSpec((E, C), _const),
                pl.BlockSpec((3 * E, E), _const),
                pl.BlockSpec((E, E), _const),
                pl.BlockSpec((3 * E, E), _const),
                pl.BlockSpec((E, E), _const),
                pl.BlockSpec((H, E), _const),
                pl.BlockSpec((E, H), _const),
                pl.BlockSpec((24, E), _const),
            ],
            out_specs=pl.BlockSpec((bb, 8, P), lambda b: (b, 0, 0)),
        ),
        compiler_params=pltpu.CompilerParams(
            dimension_semantics=("parallel",),
            vmem_limit_bytes=_VMEM_LIMIT),
    )(px, pos_embedding, pixel_embedding_w,
      l0_inter_att_in_w, l0_inter_att_out_w,
      l0_intra_att_in_w, l0_intra_att_out_w,
      l0_ff_w1, l0_ff_w2, small)

    return (y[:, :C] + embedding2pixel_b.reshape(1, C, 1)).reshape(
        B, C, Himg, Wimg)


# speedup vs baseline: 1.2058x; 1.2058x over previous
import functools

import jax
import jax.numpy as jnp
from jax.experimental import pallas as pl
from jax.experimental.pallas import tpu as pltpu

_LN_EPS = 1e-5
_VMEM_LIMIT = 56 * 1024 * 1024
_CENTRE = 4
_HEADS = 8


def _layernorm(x, g, b):
    mu = jnp.mean(x, axis=-1, keepdims=True)
    var = jnp.mean(jnp.square(x - mu), axis=-1, keepdims=True)
    return (x - mu) * jax.lax.rsqrt(var + _LN_EPS) * g + b


def _gelu(x):
    a1, a2, a3, a4, a5 = (0.254829592, -0.284496736, 1.421413741,
                          -1.453152027, 1.061405429)
    pc = 0.3275911
    z = x * 0.7071067811865476
    az = jnp.abs(z)
    t = pl.reciprocal(1.0 + pc * az, approx=True)
    poly = ((((a5 * t + a4) * t + a3) * t + a2) * t + a1) * t
    erf_abs = 1.0 - poly * jnp.exp(-az * az)
    erf = jnp.where(z < 0.0, -erf_abs, erf_abs)
    return 0.5 * x * (1.0 + erf)


def _dot_tb(x, w):
    return jax.lax.dot_general(x, w, (((1,), (1,)), ((), ())),
                               preferred_element_type=jnp.float32)


def _inter_block(px, pos_ref, wpe, wq, wkv, wo,
                 bpe, g_in, b_ln, bq, bkv_b, bo,
                 *, n_nb, centre, heads):
    PN = px.shape[1]
    E = wpe.shape[0]
    P = PN // n_nb
    d = E // heads
    scale = 1.0 / (d ** 0.5)

    lane = jax.lax.broadcasted_iota(jnp.int32, (E, heads), 0)
    head = jax.lax.broadcasted_iota(jnp.int32, (E, heads), 1)
    hsel = (lane // d == head).astype(jnp.float32)

    emb = jax.lax.dot_general(px.astype(jnp.bfloat16), wpe,
                              (((0,), (1,)), ((), ())),
                              preferred_element_type=jnp.float32) + bpe
    x3 = emb.astype(jnp.bfloat16).reshape(n_nb, P, E) \
        + pos_ref[0, :n_nb].astype(jnp.bfloat16)[:, None, :]

    xf = x3.astype(jnp.float32).reshape(PN, E)
    xn = _layernorm(xf, g_in, b_ln)

    kv = _dot_tb(xn.astype(jnp.bfloat16), wkv) + bkv_b
    xc = xn.reshape(n_nb, P, E)[centre]
    q = _dot_tb(xc.astype(jnp.bfloat16), wq) + bq

    kv3 = kv.reshape(n_nb, P, 2 * E)
    k3 = kv3[:, :, :E]
    v3 = kv3[:, :, E:]

    s_all = q[None, :, :] * k3
    s_h = jnp.dot(s_all.reshape(PN, E), hsel,
                  preferred_element_type=jnp.float32) * scale
    s_h = s_h.reshape(n_nb, P, heads)

    m = jnp.max(s_h, axis=0, keepdims=True)
    p = jnp.exp(s_h - m)
    den = jnp.sum(p, axis=0, keepdims=True)
    p = p * pl.reciprocal(den, approx=True)

    p_full = jax.lax.dot_general(p.reshape(PN, heads), hsel,
                                 (((1,), (1,)), ((), ())),
                                 preferred_element_type=jnp.float32)
    ctx = jnp.sum(p_full.reshape(n_nb, P, E) * v3, axis=0)

    out = _dot_tb(ctx.astype(jnp.bfloat16), wo) + bo
    out = out + xf.reshape(n_nb, P, E)[centre]
    return out.astype(jnp.bfloat16)


def _full_kernel(px_ref, pos_ref, wpe_ref, w_in_ref, wo_ref,
                 w_in2_ref, wo2_ref, w1_ref, w2_ref, s_ref, o_ref,
                 *, n_nb, centre, heads, bb):
    bf16 = jnp.bfloat16
    E = wpe_ref.shape[0]
    s = s_ref[...]
    wpe = wpe_ref[...].astype(bf16)
    wq = w_in_ref[:E].astype(bf16)
    wkv = w_in_ref[E:].astype(bf16)
    wo = wo_ref[...].astype(bf16)
    w_in2 = w_in2_ref[...].astype(bf16)
    wo2 = wo2_ref[...].astype(bf16)
    w1 = w1_ref[...].astype(bf16)
    w2 = w2_ref[...].astype(bf16)
    bkv_b = jnp.concatenate([s[4:5], s[5:6]], axis=1)
    b_in2 = jnp.concatenate([s[7:8], s[8:9], s[9:10]], axis=1)
    for i in range(bb):
        centre_rows = _inter_block(
            px_ref[i], pos_ref, wpe, wq, wkv, wo,
            s[0:1], s[1:2], s[2:3], s[3:4], bkv_b, s[6:7],
            n_nb=n_nb, centre=centre, heads=heads)
        _intra_block(centre_rows, w_in2, wo2, w1, w2,
                     b_in2, s[10:11], s[11:12], s[12:13], s[13:14],
                     s[14:15], s[15:16], s[16:17], s[17:20], o_ref, i,
                     heads=heads)


def _intra_block(x_in, w_in2, wo2, w1, w2,
                 b_in, bo, ffg, ffb, b1, b2, fg, fb, we, o_ref, oi, *, heads):
    x = x_in.astype(jnp.float32)
    P, E = x.shape
    d = E // heads
    scale = 1.0 / (d ** 0.5)

    qkv = _dot_tb(x.astype(jnp.bfloat16), w_in2) + b_in

    ctx = []
    for h in range(heads):
        lo = h * d
        q_h = qkv[:, lo:lo + d].astype(jnp.bfloat16)
        k_h = qkv[:, E + lo:E + lo + d].astype(jnp.bfloat16)
        v_h = qkv[:, 2 * E + lo:2 * E + lo + d].astype(jnp.bfloat16)
        s = jax.lax.dot_general(q_h, k_h, (((1,), (1,)), ((), ())),
                                preferred_element_type=jnp.float32) * scale
        m = jnp.max(s, axis=-1, keepdims=True)
        p = jnp.exp(s - m)
        den = jnp.sum(p, axis=-1, keepdims=True)
        attn = p * pl.reciprocal(den, approx=True)
        ctx.append(jnp.dot(attn.astype(jnp.bfloat16), v_h,
                           preferred_element_type=jnp.float32))
    ctx = jnp.concatenate(ctx, axis=-1)

    att = _dot_tb(ctx.astype(jnp.bfloat16), wo2) + bo
    y = att + x

    yn = _layernorm(y, ffg, ffb)
    h1 = _dot_tb(yn.astype(jnp.bfloat16), w1) + b1
    h1 = _gelu(h1)
    h2 = _dot_tb(h1.astype(jnp.bfloat16), w2) + b2
    z = (h2 + y).astype(jnp.bfloat16).astype(jnp.float32)

    zn = _layernorm(z, fg, fb)
    we8 = jnp.concatenate(
        [we.astype(jnp.bfloat16),
         jnp.zeros((8 - we.shape[0], E), jnp.bfloat16)], axis=0)
    out_t = jax.lax.dot_general(we8, zn.astype(jnp.bfloat16),
                                (((1,), (1,)), ((), ())),
                                preferred_element_type=jnp.float32)
    o_ref[oi] = out_t


def kernel(img, pixel_embedding_w, pixel_embedding_b, pos_embedding,
           final_ln_g, final_ln_b, embedding2pixel_w, embedding2pixel_b,
           l0_inter_ln_g, l0_inter_ln_b, l0_inter_att_in_w, l0_inter_att_in_b,
           l0_inter_att_out_w, l0_inter_att_out_b,
           l0_intra_att_in_w, l0_intra_att_in_b, l0_intra_att_out_w,
           l0_intra_att_out_b, l0_ff_ln_g, l0_ff_ln_b, l0_ff_w1, l0_ff_b1,
           l0_ff_w2, l0_ff_b2):
    B, C, N, Himg, Wimg = img.shape
    P = Himg * Wimg
    E = pos_embedding.shape[-1]
    heads = _HEADS
    H = l0_ff_w1.shape[0]

    f32, bf16 = jnp.float32, jnp.bfloat16

    px = img.reshape(B, C, N * P)

    small = jnp.concatenate([
        pixel_embedding_b.reshape(1, E),
        l0_inter_ln_g.reshape(1, E),
        l0_inter_ln_b.reshape(1, E),
        l0_inter_att_in_b.reshape(3, E),
        l0_inter_att_out_b.reshape(1, E),
        l0_intra_att_in_b.reshape(3, E),
        l0_intra_att_out_b.reshape(1, E),
        l0_ff_ln_g.reshape(1, E),
        l0_ff_ln_b.reshape(1, E),
        l0_ff_b1.reshape(1, H),
        l0_ff_b2.reshape(1, E),
        final_ln_g.reshape(1, E),
        final_ln_b.reshape(1, E),
        embedding2pixel_w,
        jnp.zeros((4, E), f32),
    ], axis=0)

    _const = lambda b: (0, 0)
    bb = 2 if B % 2 == 0 else 1
    kern = functools.partial(_full_kernel, n_nb=N, centre=_CENTRE,
                             heads=heads, bb=bb)
    y = pl.pallas_call(
        kern,
        out_shape=jax.ShapeDtypeStruct((B, 8, P), f32),
        grid_spec=pltpu.PrefetchScalarGridSpec(
            num_scalar_prefetch=0,
            grid=(B // bb,),
            in_specs=[
                pl.BlockSpec((bb, C, N * P), lambda b: (b, 0, 0)),
                pl.BlockSpec((1, P, E), lambda b: (0, 0, 0)),
                pl.BlockSpec((E, C), _const),
                pl.BlockSpec((3 * E, E), _const),
                pl.BlockSpec((E, E), _const),
                pl.BlockSpec((3 * E, E), _const),
                pl.BlockSpec((E, E), _const),
                pl.BlockSpec((H, E), _const),
                pl.BlockSpec((E, H), _const),
                pl.BlockSpec((24, E), _const),
            ],
            out_specs=pl.BlockSpec((bb, 8, P), lambda b: (b, 0, 0)),
        ),
        compiler_params=pltpu.CompilerParams(
            dimension_semantics=("parallel",),
            vmem_limit_bytes=_VMEM_LIMIT),
    )(px, pos_embedding, pixel_embedding_w,
      l0_inter_att_in_w, l0_inter_att_out_w,
      l0_intra_att_in_w, l0_intra_att_out_w,
      l0_ff_w1, l0_ff_w2, small)

    return (y[:, :C] + embedding2pixel_b.reshape(1, C, 1)).reshape(
        B, C, Himg, Wimg)
